# baseline (speedup 1.0000x reference)
import functools

import jax
import jax.numpy as jnp
import numpy as np
from jax import lax
from jax.experimental import pallas as pl
from jax.experimental.pallas import tpu as pltpu
from jax.experimental.pallas import tpu_sc as plsc

_VMIN, _VMAX, _BINS = 0.0, 8.0, 64
_GAMMAS = (100.0, 10.0, 1.0)
_D = 128
_E = 320000
_GW = 3 * _D
_RBF_W = 3 * _BINS
_W = _GW + _RBF_W

_NC, _NS, _L = 2, 16, 16
_NW = _NC * _NS
_PER_W = _E // _NW
_CHUNK = 80
_N_CHUNKS = _PER_W // _CHUNK
_N_PAIRS = (_N_CHUNKS - 1) // 2


def _sc_kernel(i0_hbm, i1_hbm, i2_hbm, feat_hbm, out_hbm,
               idx_v, g_v, si0, si1, sg0, sg1, sw0, sw1):
    sem_i = (si0, si1)
    sem_g = (sg0, sg1)
    sem_w = (sw0, sw1)
    i_hbm = (i0_hbm, i1_hbm, i2_hbm)
    wid = lax.axis_index("s") * _NC + lax.axis_index("c")
    w0 = wid * _PER_W

    def stage_idx(c, s):
        base = w0 + c * _CHUNK
        for j in range(3):
            pltpu.async_copy(i_hbm[j].at[pl.ds(base, _CHUNK)],
                             idx_v.at[s, j], sem_i[s])

    def wait_idx(c, s):
        base = w0 + c * _CHUNK
        for j in range(3):
            pltpu.make_async_copy(i_hbm[j].at[pl.ds(base, _CHUNK)],
                                  idx_v.at[s, j], sem_i[s]).wait()

    def start_gathers(s):
        for j in range(3):
            pltpu.async_copy(feat_hbm.at[idx_v.at[s, j]], g_v.at[s, j],
                             sem_g[s])

    def wait_gathers(s):
        for j in range(3):
            pltpu.make_async_copy(feat_hbm.at[idx_v.at[s, j]], g_v.at[s, j],
                                  sem_g[s]).wait()

    def issue_writes(c, s):
        base = w0 + c * _CHUNK
        rows = out_hbm.at[pl.ds(base, _CHUNK)]
        for j in range(3):
            pltpu.async_copy(g_v.at[s, j], rows.at[:, pl.ds(j * _D, _D)],
                             sem_w[s])

    def wait_writes(c, s):
        base = w0 + c * _CHUNK
        rows = out_hbm.at[pl.ds(base, _CHUNK)]
        for j in range(3):
            pltpu.make_async_copy(g_v.at[s, j], rows.at[:, pl.ds(j * _D, _D)],
                                  sem_w[s]).wait()

    stage_idx(0, 0)
    wait_idx(0, 0)
    start_gathers(0)
    stage_idx(1, 1)

    def pair_body(k, carry):
        for b in range(2):
            cur, nxt = b, 1 - b
            c = 2 * k + b
            if b == 0:
                @pl.when(k > 0)
                def _():
                    wait_writes(c - 1, nxt)
            else:
                wait_writes(c - 1, nxt)
            wait_idx(c + 1, nxt)
            start_gathers(nxt)
            wait_gathers(cur)
            if b == 0:
                stage_idx(c + 2, cur)
            else:
                @pl.when(k < _N_PAIRS - 1)
                def _():
                    stage_idx(c + 2, cur)
            issue_writes(c, cur)
        return carry

    lax.fori_loop(0, _N_PAIRS, pair_body, 0)

    last = _N_CHUNKS - 1
    wait_gathers(0)
    issue_writes(last, 0)
    wait_writes(last - 1, 1)
    wait_writes(last, 0)


def _sc_gather(i0, i1, i2, FEATURE):
    mesh = plsc.VectorSubcoreMesh(
        core_axis_name="c", subcore_axis_name="s",
        num_cores=_NC, num_subcores=_NS)
    f = pl.kernel(
        _sc_kernel,
        out_type=jax.ShapeDtypeStruct((_E, _GW), jnp.float32),
        mesh=mesh,
        scratch_types=[
            pltpu.VMEM((2, 3, _CHUNK), jnp.int32),
            pltpu.VMEM((2, 3, _CHUNK, _D), jnp.float32),
            pltpu.SemaphoreType.DMA,
            pltpu.SemaphoreType.DMA,
            pltpu.SemaphoreType.DMA,
            pltpu.SemaphoreType.DMA,
            pltpu.SemaphoreType.DMA,
            pltpu.SemaphoreType.DMA,
        ],
    )
    return f(i0, i1, i2, FEATURE)


_TC_BE = 1280
_TC_NB = _E // _TC_BE
_TC_NP = _TC_NB // 2


def _tc_assemble_kernel(d_any, g_any, out_any, d_v, rbf_v, sb, po0, po1):
    sem_out = (po0, po1)
    big = pltpu.make_async_copy(g_any, out_any.at[:, pl.ds(0, _GW)], sb)
    big.start()
    pltpu.sync_copy(d_any, d_v)

    r = lax.broadcasted_iota(jnp.int32, (1, _RBF_W), 1)
    cen = (r % _BINS).astype(jnp.float32) * ((_VMAX - _VMIN) / (_BINS - 1))
    band = r // _BINS
    gam = jnp.where(band == 0, _GAMMAS[0],
                    jnp.where(band == 1, _GAMMAS[1], _GAMMAS[2]))

    def compute(i, s):
        d = d_v[pl.ds(i * _TC_BE, _TC_BE)].reshape(_TC_BE, 1)
        t = d - cen
        rbf_v[s, :, :] = jnp.exp(t * t * (-gam))

    def out_copy(i, s):
        pltpu.async_copy(
            rbf_v.at[s],
            out_any.at[pl.ds(i * _TC_BE, _TC_BE), pl.ds(_GW, _RBF_W)],
            sem_out[s])

    def wait_out(i, s):
        pltpu.make_async_copy(
            rbf_v.at[s],
            out_any.at[pl.ds(i * _TC_BE, _TC_BE), pl.ds(_GW, _RBF_W)],
            sem_out[s]).wait()

    compute(0, 0)
    out_copy(0, 0)
    compute(1, 1)
    out_copy(1, 1)

    def pair_body(k, carry):
        for b in range(2):
            i = 2 * k + b
            wait_out(i - 2, b)
            compute(i, b)
            out_copy(i, b)
        return carry

    lax.fori_loop(1, _TC_NP, pair_body, 0)
    wait_out(_TC_NB - 2, 0)
    wait_out(_TC_NB - 1, 1)
    big.wait()


def _tc_assemble(d, g):
    return pl.pallas_call(
        _tc_assemble_kernel,
        out_shape=jax.ShapeDtypeStruct((_E, _W), jnp.float32),
        in_specs=[
            pl.BlockSpec(memory_space=pl.ANY),
            pl.BlockSpec(memory_space=pl.ANY),
        ],
        out_specs=pl.BlockSpec(memory_space=pl.ANY),
        scratch_shapes=[
            pltpu.VMEM((_E,), jnp.float32),
            pltpu.VMEM((2, _TC_BE, _RBF_W), jnp.float32),
            pltpu.SemaphoreType.DMA,
            pltpu.SemaphoreType.DMA,
            pltpu.SemaphoreType.DMA,
        ],
    )(d, g)


@jax.jit
def _rbf_triangle(distance, FEATURE):
    idx = distance[:, :3].astype(jnp.int32)
    d = distance[:, 3]
    g = _sc_gather(idx[:, 0], idx[:, 1], idx[:, 2], FEATURE)
    return _tc_assemble(d, g)


def kernel(distance, FEATURE):
    return _rbf_triangle(distance, FEATURE)


# speedup vs baseline: 11.2486x; 11.2486x over previous
import functools

import jax
import jax.numpy as jnp
import numpy as np
from jax import lax
from jax.experimental import pallas as pl
from jax.experimental.pallas import tpu as pltpu
from jax.experimental.pallas import tpu_sc as plsc

_VMIN, _VMAX, _BINS = 0.0, 8.0, 64
_GAMMAS = (100.0, 10.0, 1.0)
_D = 128
_E = 320000
_GW = 3 * _D
_RBF_W = 3 * _BINS
_W = _GW + _RBF_W

_NC, _NS, _L = 2, 16, 16
_NW = _NC * _NS
_PER_W = _E // _NW
_CHUNK = 80
_N_CHUNKS = _PER_W // _CHUNK
_N_PAIRS = (_N_CHUNKS - 1) // 2


_TC_BE = 2560


def _tc_rbf_kernel(d_ref, out_ref):
    i = pl.program_id(0)
    r = lax.broadcasted_iota(jnp.int32, (1, _RBF_W), 1)
    cen = (r % _BINS).astype(jnp.float32) * ((_VMAX - _VMIN) / (_BINS - 1))
    band = r // _BINS
    gam = jnp.where(band == 0, _GAMMAS[0],
                    jnp.where(band == 1, _GAMMAS[1], _GAMMAS[2]))
    d = d_ref[pl.ds(i * _TC_BE, _TC_BE)].reshape(_TC_BE, 1)
    t = d - cen
    out_ref[:, :] = jnp.exp(t * t * (-gam))


def _tc_rbf(d):
    return pl.pallas_call(
        _tc_rbf_kernel,
        out_shape=jax.ShapeDtypeStruct((_E, _RBF_W), jnp.float32),
        grid=(_E // _TC_BE,),
        in_specs=[
            pl.BlockSpec((_E,), lambda i: (0,)),
        ],
        out_specs=pl.BlockSpec((_TC_BE, _RBF_W), lambda i: (i, 0)),
    )(d)


def _sc_kernel(i0_hbm, i1_hbm, i2_hbm, rbf_hbm, feat_hbm, out_hbm,
               idx_v, g_v, rbf_v, si0, si1, sr0, sr1, sg0, sg1, sw0, sw1):
    sem_i = (si0, si1)
    sem_r = (sr0, sr1)
    sem_g = (sg0, sg1)
    sem_w = (sw0, sw1)
    i_hbm = (i0_hbm, i1_hbm, i2_hbm)
    wid = lax.axis_index("s") * _NC + lax.axis_index("c")
    w0 = wid * _PER_W

    def stage_idx(c, s):
        base = w0 + c * _CHUNK
        for j in range(3):
            pltpu.async_copy(i_hbm[j].at[pl.ds(base, _CHUNK)],
                             idx_v.at[s, j], sem_i[s])

    def wait_idx(c, s):
        base = w0 + c * _CHUNK
        for j in range(3):
            pltpu.make_async_copy(i_hbm[j].at[pl.ds(base, _CHUNK)],
                                  idx_v.at[s, j], sem_i[s]).wait()

    def stage_rbf(c, s):
        base = w0 + c * _CHUNK
        pltpu.async_copy(rbf_hbm.at[pl.ds(base, _CHUNK)], rbf_v.at[s],
                         sem_r[s])

    def wait_rbf(c, s):
        base = w0 + c * _CHUNK
        pltpu.make_async_copy(rbf_hbm.at[pl.ds(base, _CHUNK)], rbf_v.at[s],
                              sem_r[s]).wait()

    def start_gathers(s):
        for j in range(3):
            pltpu.async_copy(feat_hbm.at[idx_v.at[s, j]], g_v.at[s, j],
                             sem_g[s])

    def wait_gathers(s):
        for j in range(3):
            pltpu.make_async_copy(feat_hbm.at[idx_v.at[s, j]], g_v.at[s, j],
                                  sem_g[s]).wait()

    def issue_writes(c, s):
        base = w0 + c * _CHUNK
        rows = out_hbm.at[pl.ds(base, _CHUNK)]
        for j in range(3):
            pltpu.async_copy(g_v.at[s, j], rows.at[:, pl.ds(j * _D, _D)],
                             sem_w[s])
        pltpu.async_copy(rbf_v.at[s, :, pl.ds(0, _D)],
                         rows.at[:, pl.ds(_GW, _D)], sem_w[s])
        pltpu.async_copy(rbf_v.at[s, :, pl.ds(_D, _RBF_W - _D)],
                         rows.at[:, pl.ds(_GW + _D, _RBF_W - _D)], sem_w[s])

    def wait_writes(c, s):
        base = w0 + c * _CHUNK
        rows = out_hbm.at[pl.ds(base, _CHUNK)]
        for j in range(3):
            pltpu.make_async_copy(g_v.at[s, j], rows.at[:, pl.ds(j * _D, _D)],
                                  sem_w[s]).wait()
        pltpu.make_async_copy(rbf_v.at[s, :, pl.ds(0, _D)],
                              rows.at[:, pl.ds(_GW, _D)], sem_w[s]).wait()
        pltpu.make_async_copy(rbf_v.at[s, :, pl.ds(_D, _RBF_W - _D)],
                              rows.at[:, pl.ds(_GW + _D, _RBF_W - _D)],
                              sem_w[s]).wait()

    stage_idx(0, 0)
    stage_rbf(0, 0)
    wait_idx(0, 0)
    start_gathers(0)
    stage_idx(1, 1)
    stage_rbf(1, 1)

    def pair_body(k, carry):
        for b in range(2):
            cur, nxt = b, 1 - b
            c = 2 * k + b
            if b == 0:
                @pl.when(k > 0)
                def _():
                    wait_writes(c - 1, nxt)
                    stage_rbf(c + 1, nxt)
            else:
                wait_writes(c - 1, nxt)
                stage_rbf(c + 1, nxt)
            wait_idx(c + 1, nxt)
            start_gathers(nxt)
            wait_gathers(cur)
            if b == 0:
                stage_idx(c + 2, cur)
            else:
                @pl.when(k < _N_PAIRS - 1)
                def _():
                    stage_idx(c + 2, cur)
            wait_rbf(c, cur)
            issue_writes(c, cur)
        return carry

    lax.fori_loop(0, _N_PAIRS, pair_body, 0)

    last = _N_CHUNKS - 1
    wait_gathers(0)
    wait_rbf(last, 0)
    issue_writes(last, 0)
    wait_writes(last - 1, 1)
    wait_writes(last, 0)


def _sc_assemble(i0, i1, i2, rbf, FEATURE):
    mesh = plsc.VectorSubcoreMesh(
        core_axis_name="c", subcore_axis_name="s",
        num_cores=_NC, num_subcores=_NS)
    f = pl.kernel(
        _sc_kernel,
        out_type=jax.ShapeDtypeStruct((_E, _W), jnp.float32),
        mesh=mesh,
        scratch_types=[
            pltpu.VMEM((2, 3, _CHUNK), jnp.int32),
            pltpu.VMEM((2, 3, _CHUNK, _D), jnp.float32),
            pltpu.VMEM((2, _CHUNK, _RBF_W), jnp.float32),
            pltpu.SemaphoreType.DMA,
            pltpu.SemaphoreType.DMA,
            pltpu.SemaphoreType.DMA,
            pltpu.SemaphoreType.DMA,
            pltpu.SemaphoreType.DMA,
            pltpu.SemaphoreType.DMA,
            pltpu.SemaphoreType.DMA,
            pltpu.SemaphoreType.DMA,
        ],
    )
    return f(i0, i1, i2, rbf, FEATURE)


@jax.jit
def _rbf_triangle(distance, FEATURE):
    idx = distance[:, :3].astype(jnp.int32)
    d = distance[:, 3]
    rbf = _tc_rbf(d)
    return _sc_assemble(idx[:, 0], idx[:, 1], idx[:, 2], rbf, FEATURE)


def kernel(distance, FEATURE):
    return _rbf_triangle(distance, FEATURE)
